# Initial kernel scaffold; baseline (speedup 1.0000x reference)
import jax
import jax.numpy as jnp
from jax.experimental import pallas as pl


def kernel(z, embeddings):
    raise NotImplementedError("write your pallas kernel here")


# speedup vs baseline: 5.0424x; 5.0424x over previous
import jax
import jax.numpy as jnp
from jax import lax
from jax.experimental import pallas as pl
from jax.experimental.pallas import tpu as pltpu
from jax.experimental.pallas import tpu_sc as plsc

_N = 1024
_K = 512
_D = 256


def _dist_argmin_body(z_ref, e_ref, idx_ref, loss_ref):
    z = z_ref[:]
    e = e_ref[:]
    g = lax.dot_general(
        z, e, (((1,), (1,)), ((), ())),
        precision=lax.Precision.HIGHEST,
        preferred_element_type=jnp.float32)
    en_row = lax.dot_general(
        jnp.ones((1, _D), jnp.float32), e * e, (((1,), (1,)), ((), ())),
        precision=lax.Precision.HIGHEST,
        preferred_element_type=jnp.float32)
    m = en_row - 2.0 * g
    mmin = jnp.min(m, axis=1, keepdims=True)
    iota = lax.broadcasted_iota(jnp.int32, (_N, _K), 1)
    idx = jnp.min(jnp.where(m <= mmin, iota, _K), axis=1, keepdims=True)
    zn = jnp.sum(z * z, axis=1, keepdims=True)
    idx_ref[...] = idx
    loss_ref[0, 0] = jnp.sum(zn + mmin) / (_N * _D)


_dist_call = pl.pallas_call(
    _dist_argmin_body,
    out_shape=(
        jax.ShapeDtypeStruct((_N, 1), jnp.int32),
        jax.ShapeDtypeStruct((1, 1), jnp.float32),
    ),
    out_specs=(
        pl.BlockSpec(memory_space=pltpu.VMEM),
        pl.BlockSpec(memory_space=pltpu.SMEM),
    ),
)


_NC = 2
_NS = 16
_NW = _NC * _NS
_BPW = _N // _NW


def _gather_body(table_hbm, idx_hbm, out_hbm, idx_v, rows_v, sem):
    wid = lax.axis_index("s") * _NC + lax.axis_index("c")
    base = wid * _BPW
    pltpu.sync_copy(idx_hbm.at[pl.ds(base, _BPW)], idx_v)
    pltpu.async_copy(table_hbm.at[idx_v], rows_v, sem).wait()
    pltpu.sync_copy(rows_v, out_hbm.at[pl.ds(base, _BPW)])


_gather = pl.kernel(
    _gather_body,
    out_type=jax.ShapeDtypeStruct((_N, _D), jnp.float32),
    mesh=plsc.VectorSubcoreMesh(core_axis_name="c", subcore_axis_name="s"),
    scratch_types=[
        pltpu.VMEM((_BPW,), jnp.int32),
        pltpu.VMEM((_BPW, _D), jnp.float32),
        pltpu.SemaphoreType.DMA,
    ],
)


def kernel(z, embeddings):
    idx2, loss2 = _dist_call(z, embeddings)
    idx = idx2.reshape(_N)
    z_q = _gather(embeddings, idx)
    z_q_st = z + (z_q - z)
    return (z_q_st, loss2[0, 0], idx)


# speedup vs baseline: 5.7725x; 1.1448x over previous
import jax
import jax.numpy as jnp
from jax import lax
from jax.experimental import pallas as pl
from jax.experimental.pallas import tpu as pltpu
from jax.experimental.pallas import tpu_sc as plsc

_N = 1024
_K = 512
_D = 256


def _dist_argmin_body(z_ref, e_ref, idx_ref, loss_ref):
    z = z_ref[:]
    e = e_ref[:]
    g = lax.dot_general(
        z, e, (((1,), (1,)), ((), ())),
        precision=lax.Precision.HIGHEST,
        preferred_element_type=jnp.float32)
    en_row = lax.dot_general(
        jnp.ones((1, _D), jnp.float32), e * e, (((1,), (1,)), ((), ())),
        precision=lax.Precision.HIGHEST,
        preferred_element_type=jnp.float32)
    m = en_row - 2.0 * g
    mmin = jnp.min(m, axis=1, keepdims=True)
    iota = lax.broadcasted_iota(jnp.int32, (_N, _K), 1)
    idx = jnp.min(jnp.where(m <= mmin, iota, _K), axis=1, keepdims=True)
    zn = jnp.sum(z * z, axis=1, keepdims=True)
    idx_ref[...] = idx.reshape(_N)
    loss_ref[0, 0] = jnp.sum(zn + mmin) / (_N * _D)


_dist_call = pl.pallas_call(
    _dist_argmin_body,
    out_shape=(
        jax.ShapeDtypeStruct((_N,), jnp.int32),
        jax.ShapeDtypeStruct((1, 1), jnp.float32),
    ),
    out_specs=(
        pl.BlockSpec(memory_space=pltpu.VMEM),
        pl.BlockSpec(memory_space=pltpu.SMEM),
    ),
)


_NC = 2
_NS = 16
_NW = _NC * _NS
_BPW = _N // _NW


def _gather_body(table_hbm, idx_hbm, out_hbm, idx_v, rows_v, sem):
    wid = lax.axis_index("s") * _NC + lax.axis_index("c")
    base = wid * _BPW
    pltpu.sync_copy(idx_hbm.at[pl.ds(base, _BPW)], idx_v)
    pltpu.async_copy(table_hbm.at[idx_v], rows_v, sem).wait()
    pltpu.sync_copy(rows_v, out_hbm.at[pl.ds(base, _BPW)])


_gather = pl.kernel(
    _gather_body,
    out_type=jax.ShapeDtypeStruct((_N, _D), jnp.float32),
    mesh=plsc.VectorSubcoreMesh(core_axis_name="c", subcore_axis_name="s"),
    scratch_types=[
        pltpu.VMEM((_BPW,), jnp.int32),
        pltpu.VMEM((_BPW, _D), jnp.float32),
        pltpu.SemaphoreType.DMA,
    ],
)


def kernel(z, embeddings):
    idx, loss2 = _dist_call(z, embeddings)
    z_q = _gather(embeddings, idx)
    return (z_q, loss2[0, 0], idx)
